# Initial kernel scaffold; baseline (speedup 1.0000x reference)
import jax
import jax.numpy as jnp
from jax.experimental import pallas as pl


def kernel(x, edge_index):
    raise NotImplementedError("write your pallas kernel here")


# speedup vs baseline: 5.2173x; 5.2173x over previous
import functools

import jax
import jax.numpy as jnp
from jax import lax
from jax.experimental import pallas as pl
from jax.experimental.pallas import tpu as pltpu
from jax.experimental.pallas import tpu_sc as plsc

NUM_CORES = 2
NUM_SUBCORES = 16
NUM_WORKERS = NUM_CORES * NUM_SUBCORES


def _gather_kernel(E, D, C, table_hbm, idx_hbm, out_hbm, idx_v, buf, sem):
    b_per_w = E // NUM_WORKERS
    n_chunks = b_per_w // C
    wid = lax.axis_index("s") * NUM_CORES + lax.axis_index("c")
    base = wid * b_per_w
    pltpu.sync_copy(idx_hbm.at[pl.ds(base, b_per_w)], idx_v)
    for c in range(n_chunks):
        pltpu.async_copy(
            table_hbm.at[idx_v.at[pl.ds(c * C, C)]], buf, sem
        ).wait()
        pltpu.sync_copy(buf, out_hbm.at[pl.ds(base + c * C, C)])


def kernel(x, edge_index):
    V, D = x.shape
    E = edge_index.shape[1]
    src = edge_index[0].astype(jnp.int32)

    b_per_w = E // NUM_WORKERS
    C = 400

    mesh = plsc.VectorSubcoreMesh(
        core_axis_name="c",
        subcore_axis_name="s",
        num_cores=NUM_CORES,
        num_subcores=NUM_SUBCORES,
    )
    k = pl.kernel(
        functools.partial(_gather_kernel, E, D, C),
        out_type=jax.ShapeDtypeStruct((E, D), jnp.float32),
        mesh=mesh,
        scratch_types=[
            pltpu.VMEM((b_per_w,), jnp.int32),
            pltpu.VMEM((C, D), jnp.float32),
            pltpu.SemaphoreType.DMA,
        ],
    )
    return k(x, src)


# speedup vs baseline: 5.7023x; 1.0930x over previous
import functools

import jax
import jax.numpy as jnp
from jax import lax
from jax.experimental import pallas as pl
from jax.experimental.pallas import tpu as pltpu
from jax.experimental.pallas import tpu_sc as plsc

NUM_CORES = 2
NUM_SUBCORES = 16
NUM_WORKERS = NUM_CORES * NUM_SUBCORES


def _gather_kernel(
    E, D, C, table_hbm, idx_hbm, out_hbm, idx_v,
    buf0, buf1, gsem0, gsem1, ssem0, ssem1,
):
    b_per_w = E // NUM_WORKERS
    n_chunks = b_per_w // C
    bufs = (buf0, buf1)
    gsems = (gsem0, gsem1)
    ssems = (ssem0, ssem1)
    wid = lax.axis_index("s") * NUM_CORES + lax.axis_index("c")
    base = wid * b_per_w
    pltpu.sync_copy(idx_hbm.at[pl.ds(base, b_per_w)], idx_v)

    def start_gather(c):
        return pltpu.async_copy(
            table_hbm.at[idx_v.at[pl.ds(c * C, C)]], bufs[c % 2], gsems[c % 2]
        )

    def start_store(c):
        return pltpu.async_copy(
            bufs[c % 2], out_hbm.at[pl.ds(base + c * C, C)], ssems[c % 2]
        )

    gathers, stores = {}, {}
    gathers[0] = start_gather(0)
    for c in range(n_chunks):
        if c + 1 < n_chunks:
            if c >= 1:
                stores[c - 1].wait()
            gathers[c + 1] = start_gather(c + 1)
        gathers[c].wait()
        stores[c] = start_store(c)
    stores[n_chunks - 2].wait()
    stores[n_chunks - 1].wait()


def kernel(x, edge_index):
    V, D = x.shape
    E = edge_index.shape[1]
    src = edge_index[0].astype(jnp.int32)

    b_per_w = E // NUM_WORKERS
    C = 400

    mesh = plsc.VectorSubcoreMesh(
        core_axis_name="c",
        subcore_axis_name="s",
        num_cores=NUM_CORES,
        num_subcores=NUM_SUBCORES,
    )
    k = pl.kernel(
        functools.partial(_gather_kernel, E, D, C),
        out_type=jax.ShapeDtypeStruct((E, D), jnp.float32),
        mesh=mesh,
        scratch_types=[
            pltpu.VMEM((b_per_w,), jnp.int32),
            pltpu.VMEM((C, D), jnp.float32),
            pltpu.VMEM((C, D), jnp.float32),
            pltpu.SemaphoreType.DMA,
            pltpu.SemaphoreType.DMA,
            pltpu.SemaphoreType.DMA,
            pltpu.SemaphoreType.DMA,
        ],
    )
    return k(x, src)
